# baseline (speedup 1.0000x reference)
import functools

import jax
import jax.numpy as jnp
from jax import lax
from jax.experimental import pallas as pl
from jax.experimental.pallas import tpu as pltpu
from jax.experimental.pallas import tpu_sc as plsc

_C = 32


@functools.lru_cache(maxsize=None)
def _make_sc_kernel(d: int, cnt_max: int, base_cnt: int, rem: int, nk: int):
    mesh = plsc.VectorSubcoreMesh(core_axis_name="c", subcore_axis_name="s",
                                  num_cores=2, num_subcores=16)
    nc = mesh.num_cores
    nw = nc * mesh.num_subcores
    nv = d // 16
    ng = _C * nk
    half = ng // 2

    @functools.partial(
        pl.kernel,
        out_type=jax.ShapeDtypeStruct((nw, 16), jnp.float32),
        mesh=mesh,
        compiler_params=pltpu.CompilerParams(use_tc_tiling_on_sc=False),
        scratch_types=[
            pltpu.VMEM((cnt_max, ng), jnp.int32),
            pltpu.VMEM((2, _C, d), jnp.float32),
            pltpu.VMEM((2, ng, d), jnp.float32),
            pltpu.VMEM((16,), jnp.float32),
            pltpu.SemaphoreType.DMA,
            pltpu.SemaphoreType.DMA,
        ],
    )
    def launch(table, idx_tiles, out, idx_v, cbuf, nbuf, accv, sem0, sem1):
        wid = lax.axis_index("s") * nc + lax.axis_index("c")
        start = wid * base_cnt + jnp.minimum(wid, rem)
        cnt = base_cnt + (wid < rem).astype(jnp.int32)
        sems = (sem0, sem1)

        pltpu.sync_copy(idx_tiles.at[pl.ds(start, cnt_max)], idx_v)
        accv[...] = jnp.zeros((16,), jnp.float32)

        def issue(j, p):
            base = (start + j) * _C
            pltpu.async_copy(table.at[pl.ds(base, _C)], cbuf.at[p], sems[p])
            for g in range(2):
                pltpu.async_copy(table.at[idx_v.at[j, pl.ds(g * half, half)]],
                                 nbuf.at[p, pl.ds(g * half, half)], sems[p])

        def wait_chunk(j, p):
            base = (start + j) * _C
            pltpu.make_async_copy(
                table.at[pl.ds(base, _C)], cbuf.at[p], sems[p]).wait()
            for g in range(2):
                pltpu.make_async_copy(
                    table.at[idx_v.at[j, pl.ds(g * half, half)]],
                    nbuf.at[p, pl.ds(g * half, half)], sems[p]).wait()

        def compute(p):
            def row_body(r, a):
                ctr = [cbuf[p, r, pl.ds(16 * v, 16)] for v in range(nv)]
                a = list(a)
                for k in range(nk):
                    for v in range(nv):
                        a[v] = a[v] + jnp.abs(
                            nbuf[p, r * nk + k, pl.ds(16 * v, 16)] - ctr[v])
                return tuple(a)

            zeros = jnp.zeros((16,), jnp.float32)
            accs = lax.fori_loop(0, _C, row_body, (zeros,) * nv)
            tot = accs[0]
            for v in range(1, nv):
                tot = tot + accs[v]
            accv[...] = accv[...] + tot

        issue(0, 0)

        def body2(jj, _):
            j0 = jj * 2

            @pl.when(j0 + 1 < cnt)
            def _():
                issue(j0 + 1, 1)

            wait_chunk(j0, 0)
            compute(0)

            @pl.when(j0 + 2 < cnt)
            def _():
                issue(j0 + 2, 0)

            @pl.when(j0 + 1 < cnt)
            def _():
                wait_chunk(j0 + 1, 1)
                compute(1)

            return 0

        lax.fori_loop(0, (cnt + 1) // 2, body2, 0)
        pltpu.sync_copy(accv, out.at[wid])

    return launch, nw


def kernel(output, nh_indices):
    b, n, d = output.shape
    k_all = nh_indices.shape[1]
    nk = k_all - 1
    rows_total = b * n
    assert rows_total % _C == 0
    nchunk = rows_total // _C

    base_cnt, rem = nchunk // 32, nchunk % 32
    cnt_max = base_cnt + (1 if rem else 0)
    launch, nw = _make_sc_kernel(d, cnt_max, base_cnt, rem, nk)

    table = output.reshape(rows_total, d)
    nh = nh_indices[:, 1:].astype(jnp.int32)
    idx = jnp.arange(b, dtype=jnp.int32)[:, None, None] * n + nh[None]
    idx = idx.reshape(nchunk, _C * nk)
    pad = nw * cnt_max - nchunk
    if pad:
        idx = jnp.concatenate(
            [idx, jnp.zeros((pad, _C * nk), jnp.int32)], axis=0)

    partials = launch(table, idx)
    return jnp.sqrt(jnp.sum(partials) / (rows_total * nk * d))


# speedup vs baseline: 1.2868x; 1.2868x over previous
import functools

import jax
import jax.numpy as jnp
from jax import lax
from jax.experimental import pallas as pl
from jax.experimental.pallas import tpu as pltpu
from jax.experimental.pallas import tpu_sc as plsc

_GN = 8


@functools.lru_cache(maxsize=None)
def _make_sc_kernel(b: int, n: int, d: int, cnt_max: int, base_cnt: int,
                    rem: int, nk: int, kpad: int):
    mesh = plsc.VectorSubcoreMesh(core_axis_name="c", subcore_axis_name="s",
                                  num_cores=2, num_subcores=16)
    nc = mesh.num_cores
    nw = nc * mesh.num_subcores
    nv = d // 16
    rows = b * _GN
    nh_rows = cnt_max * _GN

    @functools.partial(
        pl.kernel,
        out_type=jax.ShapeDtypeStruct((nw, 16), jnp.float32),
        mesh=mesh,
        compiler_params=pltpu.CompilerParams(use_tc_tiling_on_sc=False,
                                             needs_layout_passes=False),
        scratch_types=[
            pltpu.VMEM((nh_rows, kpad), jnp.int32),
            pltpu.VMEM((2, nk, rows), jnp.int32),
            pltpu.VMEM((2, rows, d), jnp.float32),
            pltpu.VMEM((2, nk, rows, d), jnp.float32),
            pltpu.VMEM((16,), jnp.float32),
            pltpu.SemaphoreType.DMA,
            pltpu.SemaphoreType.DMA,
        ],
    )
    def launch(table, nh8, out, nh_v, idxb, cbuf, nbuf, accv, sem0, sem1):
        wid = lax.axis_index("s") * nc + lax.axis_index("c")
        start = wid * base_cnt + jnp.minimum(wid, rem)
        cnt = base_cnt + (wid < rem).astype(jnp.int32)
        sems = (sem0, sem1)

        pltpu.sync_copy(nh8.at[pl.ds(start * _GN, nh_rows)], nh_v)
        accv[...] = jnp.zeros((16,), jnp.float32)

        iota = lax.iota(jnp.int32, 16)
        jmod = iota & (_GN - 1)
        boffs = [((iota >> 3) + 2 * h) * n for h in range(rows // 16)]

        def issue(j, p):
            lj = j * _GN
            for k in range(nk):
                col = jnp.full((16,), k + 1, jnp.int32)
                for h in range(rows // 16):
                    vals = plsc.load_gather(nh_v, [lj + jmod, col])
                    idxb[p, k, pl.ds(h * 16, 16)] = vals + boffs[h]
            for bb in range(b):
                pltpu.async_copy(
                    table.at[pl.ds(bb * n + (start + j) * _GN, _GN)],
                    cbuf.at[p, pl.ds(bb * _GN, _GN)], sems[p])
            for k in range(nk):
                pltpu.async_copy(table.at[idxb.at[p, k]], nbuf.at[p, k],
                                 sems[p])

        def wait_chunk(j, p):
            for bb in range(b):
                pltpu.make_async_copy(
                    table.at[pl.ds(bb * n + (start + j) * _GN, _GN)],
                    cbuf.at[p, pl.ds(bb * _GN, _GN)], sems[p]).wait()
            for k in range(nk):
                pltpu.make_async_copy(
                    table.at[idxb.at[p, k]], nbuf.at[p, k], sems[p]).wait()

        def compute(p):
            def row_body(r, a):
                ctr = [cbuf[p, r, pl.ds(16 * v, 16)] for v in range(nv)]
                a = list(a)
                for k in range(nk):
                    for v in range(nv):
                        a[v] = a[v] + jnp.abs(
                            nbuf[p, k, r, pl.ds(16 * v, 16)] - ctr[v])
                return tuple(a)

            zeros = jnp.zeros((16,), jnp.float32)
            accs = lax.fori_loop(0, rows, row_body, (zeros,) * nv)
            tot = accs[0]
            for v in range(1, nv):
                tot = tot + accs[v]
            accv[...] = accv[...] + tot

        issue(0, 0)

        def body2(jj, _):
            j0 = jj * 2

            @pl.when(j0 + 1 < cnt)
            def _():
                issue(j0 + 1, 1)

            wait_chunk(j0, 0)
            compute(0)

            @pl.when(j0 + 2 < cnt)
            def _():
                issue(j0 + 2, 0)

            @pl.when(j0 + 1 < cnt)
            def _():
                wait_chunk(j0 + 1, 1)
                compute(1)

            return 0

        lax.fori_loop(0, (cnt + 1) // 2, body2, 0)
        pltpu.sync_copy(accv, out.at[wid])

    return launch, nw


def kernel(output, nh_indices):
    b, n, d = output.shape
    k_all = nh_indices.shape[1]
    nk = k_all - 1
    assert n % _GN == 0
    ngroups = n // _GN

    base_cnt, rem = ngroups // 32, ngroups % 32
    cnt_max = base_cnt + (1 if rem else 0)
    kpad = 8
    launch, nw = _make_sc_kernel(b, n, d, cnt_max, base_cnt, rem, nk, kpad)

    table = output.reshape(b * n, d)
    row_pad = (nw - 1) * base_cnt + min(nw - 1, rem) + cnt_max
    row_pad = row_pad * _GN - n
    nh8 = jnp.pad(nh_indices.astype(jnp.int32),
                  ((0, max(row_pad, 0)), (0, kpad - k_all)))

    partials = launch(table, nh8)
    return jnp.sqrt(jnp.sum(partials) / (b * n * nk * d))


# speedup vs baseline: 1.5631x; 1.2147x over previous
import functools

import jax
import jax.numpy as jnp
from jax import lax
from jax.experimental import pallas as pl
from jax.experimental.pallas import tpu as pltpu
from jax.experimental.pallas import tpu_sc as plsc

_GN = 8


@functools.lru_cache(maxsize=None)
def _make_sc_kernel(b: int, n: int, d: int, cnt_max: int, base_cnt: int,
                    rem: int, nk: int, kpad: int):
    mesh = plsc.VectorSubcoreMesh(core_axis_name="c", subcore_axis_name="s",
                                  num_cores=2, num_subcores=16)
    nc = mesh.num_cores
    nw = nc * mesh.num_subcores
    nv = d // 16
    rows = b * _GN
    nh_rows = cnt_max * _GN

    @functools.partial(
        pl.kernel,
        out_type=jax.ShapeDtypeStruct((nw, 16), jnp.float32),
        mesh=mesh,
        compiler_params=pltpu.CompilerParams(use_tc_tiling_on_sc=False,
                                             needs_layout_passes=False),
        scratch_types=[
            pltpu.VMEM((nh_rows, kpad), jnp.int32),
            pltpu.VMEM((3, nk, rows), jnp.int32),
            pltpu.VMEM((3, rows, d), jnp.float32),
            pltpu.VMEM((3, nk, rows, d), jnp.float32),
            pltpu.VMEM((16,), jnp.float32),
            pltpu.SemaphoreType.DMA,
            pltpu.SemaphoreType.DMA,
            pltpu.SemaphoreType.DMA,
        ],
    )
    def launch(table, nh8, out, nh_v, idxb, cbuf, nbuf, accv,
               sem0, sem1, sem2):
        wid = lax.axis_index("s") * nc + lax.axis_index("c")
        start = wid * base_cnt + jnp.minimum(wid, rem)
        cnt = base_cnt + (wid < rem).astype(jnp.int32)
        sems = (sem0, sem1, sem2)

        pltpu.sync_copy(nh8.at[pl.ds(start * _GN, nh_rows)], nh_v)
        accv[...] = jnp.zeros((16,), jnp.float32)

        iota = lax.iota(jnp.int32, 16)
        jmod = iota & (_GN - 1)
        boffs = [((iota >> 3) + 2 * h) * n for h in range(rows // 16)]

        def issue(j, p):
            lj = j * _GN
            for k in range(nk):
                col = jnp.full((16,), k + 1, jnp.int32)
                for h in range(rows // 16):
                    vals = plsc.load_gather(nh_v, [lj + jmod, col])
                    idxb[p, k, pl.ds(h * 16, 16)] = vals + boffs[h]
            for bb in range(b):
                pltpu.async_copy(
                    table.at[pl.ds(bb * n + (start + j) * _GN, _GN)],
                    cbuf.at[p, pl.ds(bb * _GN, _GN)], sems[p])
            for k in range(nk):
                pltpu.async_copy(table.at[idxb.at[p, k]], nbuf.at[p, k],
                                 sems[p])

        def wait_chunk(j, p):
            for bb in range(b):
                pltpu.make_async_copy(
                    table.at[pl.ds(bb * n + (start + j) * _GN, _GN)],
                    cbuf.at[p, pl.ds(bb * _GN, _GN)], sems[p]).wait()
            for k in range(nk):
                pltpu.make_async_copy(
                    table.at[idxb.at[p, k]], nbuf.at[p, k], sems[p]).wait()

        def compute(p):
            def row_body(r, a):
                ctr = [cbuf[p, r, pl.ds(16 * v, 16)] for v in range(nv)]
                a = list(a)
                for k in range(nk):
                    for v in range(nv):
                        a[v] = a[v] + jnp.abs(
                            nbuf[p, k, r, pl.ds(16 * v, 16)] - ctr[v])
                return tuple(a)

            zeros = jnp.zeros((16,), jnp.float32)
            accs = lax.fori_loop(0, rows, row_body, (zeros,) * nv)
            tot = accs[0]
            for v in range(1, nv):
                tot = tot + accs[v]
            accv[...] = accv[...] + tot

        issue(0, 0)

        @pl.when(1 < cnt)
        def _():
            issue(1, 1)

        def body3(jj, _):
            j0 = jj * 3
            for p in range(3):
                j = j0 + p

                @pl.when(j + 2 < cnt)
                def _():
                    issue(j + 2, (p + 2) % 3)

                @pl.when(j < cnt)
                def _():
                    wait_chunk(j, p)
                    compute(p)

            return 0

        lax.fori_loop(0, (cnt + 2) // 3, body3, 0)
        pltpu.sync_copy(accv, out.at[wid])

    return launch, nw


def kernel(output, nh_indices):
    b, n, d = output.shape
    k_all = nh_indices.shape[1]
    nk = k_all - 1
    assert n % _GN == 0
    ngroups = n // _GN

    base_cnt, rem = ngroups // 32, ngroups % 32
    cnt_max = base_cnt + (1 if rem else 0)
    kpad = 8
    launch, nw = _make_sc_kernel(b, n, d, cnt_max, base_cnt, rem, nk, kpad)

    table = output.reshape(b * n, d)
    row_pad = (nw - 1) * base_cnt + min(nw - 1, rem) + cnt_max
    row_pad = row_pad * _GN - n
    nh8 = jnp.pad(nh_indices.astype(jnp.int32),
                  ((0, max(row_pad, 0)), (0, kpad - k_all)))

    partials = launch(table, nh8)
    return jnp.sqrt(jnp.sum(partials) / (b * n * nk * d))
